# baseline (speedup 1.0000x reference)
import functools

import jax
import jax.numpy as jnp
from jax import lax
from jax.experimental import pallas as pl
from jax.experimental.pallas import tpu as pltpu
from jax.experimental.pallas import tpu_sc as plsc

BATCH = 4096
SEQ_LEN = 200
EMBED_DIM = 64
TOTAL = BATCH * SEQ_LEN

NC = 2
NS = 16
NW = NC * NS

L = 16
G = 8
BLK = G * L
PER_W = TOTAL // NW
NBLK = PER_W // BLK
NBUF = 4
KLAG = 2

assert PER_W * NW == TOTAL and NBLK * BLK == PER_W
assert NBLK % NBUF == 0 and 0 < KLAG < NBUF

_mesh = plsc.VectorSubcoreMesh(core_axis_name="c", subcore_axis_name="s")


@functools.partial(
    pl.kernel,
    out_type=jax.ShapeDtypeStruct((TOTAL, EMBED_DIM), jnp.float32),
    mesh=_mesh,
    scratch_types=[
        pltpu.VMEM((NBLK * G, L), jnp.int32),
        pltpu.VMEM((NBUF, BLK, EMBED_DIM), jnp.float32),
        pltpu.SemaphoreType.DMA((NBUF,)),
        pltpu.SemaphoreType.DMA((NBUF,)),
    ],
    compiler_params=pltpu.CompilerParams(use_tc_tiling_on_sc=False),
)
def _gather_kernel(ids_hbm, table_hbm, out_hbm, idx_v, rows_v, g_sem, s_sem):
    wid = lax.axis_index("s") * NC + lax.axis_index("c")
    out_base = wid * PER_W

    pltpu.sync_copy(ids_hbm.at[wid], idx_v)

    def start_gathers(c, b):
        for g in range(G):
            iv = idx_v[c * G + g]
            pltpu.async_copy(
                table_hbm.at[iv], rows_v.at[b, pl.ds(g * L, L)], g_sem.at[b]
            )

    def wait_gathers(c, b):
        for g in range(G):
            iv = idx_v[c * G + g]
            pltpu.make_async_copy(
                table_hbm.at[iv], rows_v.at[b, pl.ds(g * L, L)], g_sem.at[b]
            ).wait()

    def start_scatter(c, b):
        pltpu.async_copy(
            rows_v.at[b], out_hbm.at[pl.ds(out_base + c * BLK, BLK)],
            s_sem.at[b],
        )

    def wait_scatter(c, b):
        pltpu.make_async_copy(
            rows_v.at[b], out_hbm.at[pl.ds(out_base + c * BLK, BLK)],
            s_sem.at[b],
        ).wait()

    for b in range(NBUF):
        start_gathers(b, b)

    @pl.loop(0, NBLK, step=NBUF)
    def _ring(q):
        for b in range(NBUF):
            c = q + b
            wait_gathers(c, b)
            start_scatter(c, b)

            b2 = (b - KLAG) % NBUF
            c2 = c - KLAG

            @pl.when((c2 >= 0) & (c2 + NBUF < NBLK))
            def _refill():
                wait_scatter(c2, b2)
                start_gathers(c2 + NBUF, b2)

    for b in range(NBUF):
        wait_scatter(NBLK - NBUF + b, b)


def kernel(source_ids, table):
    ids = source_ids.astype(jnp.int32).reshape(NW, NBLK * G, L)
    out = _gather_kernel(ids, table)
    return out.reshape(BATCH, SEQ_LEN, EMBED_DIM)


# speedup vs baseline: 1.0006x; 1.0006x over previous
import functools

import jax
import jax.numpy as jnp
from jax import lax
from jax.experimental import pallas as pl
from jax.experimental.pallas import tpu as pltpu
from jax.experimental.pallas import tpu_sc as plsc

BATCH = 4096
SEQ_LEN = 200
EMBED_DIM = 64
TOTAL = BATCH * SEQ_LEN

NC = 2
NS = 16
NW = NC * NS

L = 16
G = 8
BLK = G * L
PER_W = TOTAL // NW
NBLK = PER_W // BLK
NBUF = 4
KLAG = 2

assert PER_W * NW == TOTAL and NBLK * BLK == PER_W
assert NBLK % NBUF == 0 and 0 < KLAG < NBUF

_mesh = plsc.VectorSubcoreMesh(core_axis_name="c", subcore_axis_name="s")


@functools.partial(
    pl.kernel,
    out_type=jax.ShapeDtypeStruct((TOTAL, EMBED_DIM), jnp.float32),
    mesh=_mesh,
    scratch_types=[
        pltpu.VMEM((NBLK, BLK), jnp.int32),
        pltpu.VMEM((NBUF, BLK, EMBED_DIM), jnp.float32),
        pltpu.SemaphoreType.DMA((NBUF,)),
        pltpu.SemaphoreType.DMA((NBUF,)),
    ],
    compiler_params=pltpu.CompilerParams(use_tc_tiling_on_sc=False),
)
def _gather_kernel(ids_hbm, table_hbm, out_hbm, idx_v, rows_v, g_sem, s_sem):
    wid = lax.axis_index("s") * NC + lax.axis_index("c")
    out_base = wid * PER_W

    pltpu.sync_copy(ids_hbm.at[pl.ds(wid * NBLK, NBLK)], idx_v)

    def start_gathers(c, b):
        for g in range(G):
            iv = idx_v[c, pl.ds(g * L, L)]
            pltpu.async_copy(
                table_hbm.at[iv], rows_v.at[b, pl.ds(g * L, L)], g_sem.at[b]
            )

    def wait_gathers(c, b):
        for g in range(G):
            iv = idx_v[c, pl.ds(g * L, L)]
            pltpu.make_async_copy(
                table_hbm.at[iv], rows_v.at[b, pl.ds(g * L, L)], g_sem.at[b]
            ).wait()

    def start_scatter(c, b):
        pltpu.async_copy(
            rows_v.at[b], out_hbm.at[pl.ds(out_base + c * BLK, BLK)],
            s_sem.at[b],
        )

    def wait_scatter(c, b):
        pltpu.make_async_copy(
            rows_v.at[b], out_hbm.at[pl.ds(out_base + c * BLK, BLK)],
            s_sem.at[b],
        ).wait()

    for b in range(NBUF):
        start_gathers(b, b)

    @pl.loop(0, NBLK, step=NBUF)
    def _ring(q):
        for b in range(NBUF):
            c = q + b
            wait_gathers(c, b)
            start_scatter(c, b)

            b2 = (b - KLAG) % NBUF
            c2 = c - KLAG

            @pl.when((c2 >= 0) & (c2 + NBUF < NBLK))
            def _refill():
                wait_scatter(c2, b2)
                start_gathers(c2 + NBUF, b2)

    for b in range(NBUF):
        wait_scatter(NBLK - NBUF + b, b)


def kernel(source_ids, table):
    ids = source_ids.astype(jnp.int32).reshape(NW * NBLK, BLK)
    out = _gather_kernel(ids, table)
    return out.reshape(BATCH, SEQ_LEN, EMBED_DIM)


# speedup vs baseline: 1.0069x; 1.0064x over previous
import functools

import jax
import jax.numpy as jnp
from jax import lax
from jax.experimental import pallas as pl
from jax.experimental.pallas import tpu as pltpu
from jax.experimental.pallas import tpu_sc as plsc

BATCH = 4096
SEQ_LEN = 200
EMBED_DIM = 64
TOTAL = BATCH * SEQ_LEN

NC = 2
NS = 16
NW = NC * NS

CHUNK = 128
PER_W = TOTAL // NW
NCHUNK = PER_W // CHUNK
NBUF = 8
KLAG = 4

assert PER_W * NW == TOTAL and NCHUNK * CHUNK == PER_W
assert NCHUNK % NBUF == 0 and 0 < KLAG < NBUF

_mesh = plsc.VectorSubcoreMesh(core_axis_name="c", subcore_axis_name="s")


@functools.partial(
    pl.kernel,
    out_type=jax.ShapeDtypeStruct((TOTAL, EMBED_DIM), jnp.float32),
    mesh=_mesh,
    scratch_types=[
        pltpu.VMEM((NCHUNK, CHUNK), jnp.int32),
        pltpu.VMEM((NBUF, CHUNK, EMBED_DIM), jnp.float32),
        pltpu.SemaphoreType.DMA((NBUF,)),
        pltpu.SemaphoreType.DMA((NBUF,)),
    ],
    compiler_params=pltpu.CompilerParams(use_tc_tiling_on_sc=False),
)
def _gather_kernel(ids_hbm, table_hbm, out_hbm, idx_v, rows_v, g_sem, s_sem):
    wid = lax.axis_index("s") * NC + lax.axis_index("c")
    out_base = wid * PER_W

    pltpu.sync_copy(ids_hbm.at[wid], idx_v)

    def start_gather(c, b):
        pltpu.async_copy(table_hbm.at[idx_v.at[c]], rows_v.at[b], g_sem.at[b])

    def wait_gather(c, b):
        pltpu.make_async_copy(
            table_hbm.at[idx_v.at[c]], rows_v.at[b], g_sem.at[b]
        ).wait()

    def start_scatter(c, b):
        pltpu.async_copy(
            rows_v.at[b], out_hbm.at[pl.ds(out_base + c * CHUNK, CHUNK)],
            s_sem.at[b],
        )

    def wait_scatter(c, b):
        pltpu.make_async_copy(
            rows_v.at[b], out_hbm.at[pl.ds(out_base + c * CHUNK, CHUNK)],
            s_sem.at[b],
        ).wait()

    for b in range(NBUF):
        start_gather(b, b)

    @pl.loop(0, NCHUNK, step=NBUF)
    def _ring(g):
        for b in range(NBUF):
            c = g + b
            wait_gather(c, b)
            start_scatter(c, b)

            b2 = (b - KLAG) % NBUF
            c2 = c - KLAG

            @pl.when((c2 >= 0) & (c2 + NBUF < NCHUNK))
            def _refill():
                wait_scatter(c2, b2)
                start_gather(c2 + NBUF, b2)

    for b in range(NBUF):
        wait_scatter(NCHUNK - NBUF + b, b)


def kernel(source_ids, table):
    ids = source_ids.astype(jnp.int32).reshape(NW, NCHUNK, CHUNK)
    out = _gather_kernel(ids, table)
    return out.reshape(BATCH, SEQ_LEN, EMBED_DIM)
